# baseline (speedup 1.0000x reference)
import jax
import jax.numpy as jnp
from jax import lax
from jax.experimental import pallas as pl
from jax.experimental.pallas import tpu as pltpu
from jax.experimental.pallas import tpu_sc as plsc

_VOCAB = 100000
_CTX = 2048
_D = 768
_BATCH = 4
_SEQ = 2048

_NC = 2
_NS = 16
_NW = _NC * _NS
_N = _BATCH * _SEQ
_PER_W = _N // _NW
_C = 32
_NCHUNK = _PER_W // _C
_LANES = 16


def _body(src_hbm, pos_hbm, emb_hbm, out_hbm,
          idx_v, rows0, rows1, rows2, pos0, pos1,
          gsem0, gsem1, gsem2, psem0, psem1, osem0, osem1, osem2):
    wid = lax.axis_index("s") * _NC + lax.axis_index("c")
    base = wid * _PER_W
    pos_base = lax.rem(base, _SEQ)

    rows_bufs = [rows0, rows1, rows2]
    pos_bufs = [pos0, pos1]
    gsems = [gsem0, gsem1, gsem2]
    psems = [psem0, psem1]
    osems = [osem0, osem1, osem2]

    pltpu.sync_copy(src_hbm.at[wid], idx_v)

    def out_slice(c):
        return out_hbm.at[pl.ds(base + c * _C, _C)]

    def issue_gather(c):
        nb = c % 3
        pltpu.async_copy(emb_hbm.at[idx_v.at[c]], rows_bufs[nb], gsems[nb])

    def issue_pos(c):
        nb = c % 2
        pltpu.async_copy(pos_hbm.at[pl.ds(pos_base + c * _C, _C)],
                         pos_bufs[nb], psems[nb])

    issue_gather(0)
    issue_pos(0)
    issue_gather(1)
    issue_pos(1)
    for c in range(_NCHUNK):
        nb = c % 3
        with jax.named_scope("gwait"):
            pltpu.make_async_copy(emb_hbm.at[idx_v.at[c]], rows_bufs[nb],
                                  gsems[nb]).wait()
            pltpu.make_async_copy(pos_hbm.at[pl.ds(pos_base + c * _C, _C)],
                                  pos_bufs[c % 2], psems[c % 2]).wait()
        rows = rows_bufs[nb]
        pos = pos_bufs[c % 2]

        def row_body(r, carry):
            for j in range(_D // _LANES):
                s = pl.ds(j * _LANES, _LANES)
                plsc.addupdate(rows.at[r, s], pos[r, s])
            return carry

        with jax.named_scope("addloop"):
            lax.fori_loop(0, _C, row_body, 0)

        if c + 2 < _NCHUNK:
            if c >= 1:
                pnb = (c - 1) % 3
                with jax.named_scope("owait"):
                    pltpu.make_async_copy(rows_bufs[pnb], out_slice(c - 1),
                                          osems[pnb]).wait()
            issue_gather(c + 2)

        pltpu.async_copy(rows, out_slice(c), osems[nb])
        if c + 2 < _NCHUNK:
            issue_pos(c + 2)

    with jax.named_scope("tailwait"):
        for c in range(_NCHUNK - 3, _NCHUNK):
            pltpu.make_async_copy(rows_bufs[c % 3], out_slice(c),
                                  osems[c % 3]).wait()


@jax.jit
def _embed(src_flat, emb_table, pos_table):
    kfn = pl.kernel(
        _body,
        out_type=jax.ShapeDtypeStruct((_N, _D), jnp.float32),
        mesh=plsc.VectorSubcoreMesh(core_axis_name="c", subcore_axis_name="s",
                                    num_cores=_NC, num_subcores=_NS),
        scratch_types=[
            pltpu.VMEM((_NCHUNK, _C), jnp.int32),
            pltpu.VMEM((_C, _D), jnp.float32),
            pltpu.VMEM((_C, _D), jnp.float32),
            pltpu.VMEM((_C, _D), jnp.float32),
            pltpu.VMEM((_C, _D), jnp.float32),
            pltpu.VMEM((_C, _D), jnp.float32),
            pltpu.SemaphoreType.DMA,
            pltpu.SemaphoreType.DMA,
            pltpu.SemaphoreType.DMA,
            pltpu.SemaphoreType.DMA,
            pltpu.SemaphoreType.DMA,
            pltpu.SemaphoreType.DMA,
            pltpu.SemaphoreType.DMA,
            pltpu.SemaphoreType.DMA,
        ],
    )
    return kfn(src_flat, pos_table, emb_table)


def kernel(src, emb_table, pos_table):
    batch, seq = src.shape
    src_flat = src.reshape(_NW, _NCHUNK, _C).astype(jnp.int32)
    out = _embed(src_flat, emb_table, pos_table)
    return out.reshape(batch, seq, _D)


# speedup vs baseline: 1.1129x; 1.1129x over previous
import jax
import jax.numpy as jnp
from jax import lax
from jax.experimental import pallas as pl
from jax.experimental.pallas import tpu as pltpu
from jax.experimental.pallas import tpu_sc as plsc

_VOCAB = 100000
_CTX = 2048
_D = 768
_BATCH = 4
_SEQ = 2048

_NC = 2
_NS = 16
_NW = _NC * _NS
_N = _BATCH * _SEQ
_PER_W = _N // _NW
_C = 32
_NCHUNK = _PER_W // _C
_LANES = 16


def _body(src_hbm, pos_hbm, emb_hbm, out_hbm,
          idx_v, rows0, rows1, pos0, pos1,
          gsem0, gsem1, psem0, psem1):
    wid = lax.axis_index("s") * _NC + lax.axis_index("c")
    g = lax.rem(wid, _NS)
    bp = wid // _NS
    pos_base = g * (2 * _PER_W // _BATCH)

    rows_bufs = [rows0, rows1]
    pos_bufs = [pos0, pos1]
    gsems = [gsem0, gsem1]
    psems = [psem0, psem1]

    pltpu.sync_copy(src_hbm.at[wid], idx_v)

    def pos_slice(h):
        return pos_hbm.at[pl.ds(pos_base + h * _C, _C)]

    def out_slice(c):
        b = (2 * bp + lax.rem(c, 2)) * _SEQ
        return out_hbm.at[pl.ds(b + pos_base + (c // 2) * _C, _C)]

    def issue_gather(c):
        nb = c % 2
        pltpu.async_copy(emb_hbm.at[idx_v.at[c]], rows_bufs[nb], gsems[nb])

    def issue_pos(h):
        pltpu.async_copy(pos_slice(h), pos_bufs[h % 2], psems[h % 2])

    issue_gather(0)
    issue_pos(0)
    for c in range(_NCHUNK):
        nb = c % 2
        h = c // 2
        pltpu.make_async_copy(emb_hbm.at[idx_v.at[c]], rows_bufs[nb],
                              gsems[nb]).wait()
        if c % 2 == 0:
            pltpu.make_async_copy(pos_slice(h), pos_bufs[h % 2],
                                  psems[h % 2]).wait()
            if h + 1 < _NCHUNK // 2:
                issue_pos(h + 1)
        if c + 1 < _NCHUNK:
            issue_gather(c + 1)

        rows = rows_bufs[nb]
        pos = pos_bufs[h % 2]

        def row_body(r, carry):
            for j in range(_D // _LANES):
                s = pl.ds(j * _LANES, _LANES)
                rows[r, s] = rows[r, s] + pos[r, s]
            return carry

        lax.fori_loop(0, _C, row_body, 0)

        pltpu.sync_copy(rows, out_slice(c))


@jax.jit
def _embed(src_flat, emb_table, pos_table):
    kfn = pl.kernel(
        _body,
        out_type=jax.ShapeDtypeStruct((_N, _D), jnp.float32),
        mesh=plsc.VectorSubcoreMesh(core_axis_name="c", subcore_axis_name="s",
                                    num_cores=_NC, num_subcores=_NS),
        scratch_types=[
            pltpu.VMEM((_NCHUNK, _C), jnp.int32),
            pltpu.VMEM((_C, _D), jnp.float32),
            pltpu.VMEM((_C, _D), jnp.float32),
            pltpu.VMEM((_C, _D), jnp.float32),
            pltpu.VMEM((_C, _D), jnp.float32),
            pltpu.SemaphoreType.DMA,
            pltpu.SemaphoreType.DMA,
            pltpu.SemaphoreType.DMA,
            pltpu.SemaphoreType.DMA,
        ],
    )
    return kfn(src_flat, pos_table, emb_table)


def kernel(src, emb_table, pos_table):
    batch, seq = src.shape
    s5 = src.reshape(2, 2, _NS, _NCHUNK // 2, _C)
    src_t = (s5.transpose(0, 2, 3, 1, 4)
             .reshape(_NW, _NCHUNK, _C).astype(jnp.int32))
    out = _embed(src_t, emb_table, pos_table)
    return out.reshape(batch, seq, _D)


# speedup vs baseline: 1.1526x; 1.0356x over previous
import jax
import jax.numpy as jnp
from jax import lax
from jax.experimental import pallas as pl
from jax.experimental.pallas import tpu as pltpu
from jax.experimental.pallas import tpu_sc as plsc

_VOCAB = 100000
_CTX = 2048
_D = 768
_BATCH = 4
_SEQ = 2048

_NC = 2
_NS = 16
_NW = _NC * _NS
_N = _BATCH * _SEQ
_PER_W = _N // _NW
_C = 32
_NCHUNK = _PER_W // _C
_LANES = 16


def _body(src_hbm, pos_hbm, emb_hbm, out_hbm,
          idx_v, rows0, rows1, pos0, pos1,
          gsem0, gsem1, psem0, psem1):
    wid = lax.axis_index("s") * _NC + lax.axis_index("c")
    pos_base = wid * (_PER_W // _BATCH)

    rows_bufs = [rows0, rows1]
    pos_bufs = [pos0, pos1]
    gsems = [gsem0, gsem1]
    psems = [psem0, psem1]

    pltpu.sync_copy(src_hbm.at[wid], idx_v)

    def pos_slice(h):
        return pos_hbm.at[pl.ds(pos_base + h * _C, _C)]

    def out_slice(c):
        b = lax.rem(c, _BATCH) * _SEQ
        return out_hbm.at[pl.ds(b + pos_base + (c // _BATCH) * _C, _C)]

    def issue_gather(c):
        nb = c % 2
        pltpu.async_copy(emb_hbm.at[idx_v.at[c]], rows_bufs[nb], gsems[nb])

    def issue_pos(h):
        pltpu.async_copy(pos_slice(h), pos_bufs[h % 2], psems[h % 2])

    issue_gather(0)
    issue_pos(0)
    for c in range(_NCHUNK):
        nb = c % 2
        h = c // _BATCH
        pltpu.make_async_copy(emb_hbm.at[idx_v.at[c]], rows_bufs[nb],
                              gsems[nb]).wait()
        if c % _BATCH == 0:
            pltpu.make_async_copy(pos_slice(h), pos_bufs[h % 2],
                                  psems[h % 2]).wait()
            if h + 1 < _NCHUNK // _BATCH:
                issue_pos(h + 1)
        if c + 1 < _NCHUNK:
            issue_gather(c + 1)

        rows = rows_bufs[nb]
        pos = pos_bufs[h % 2]

        def row_body(r, carry):
            for j in range(_D // _LANES):
                s = pl.ds(j * _LANES, _LANES)
                rows[r, s] = rows[r, s] + pos[r, s]
            return carry

        lax.fori_loop(0, _C, row_body, 0)

        pltpu.sync_copy(rows, out_slice(c))


@jax.jit
def _embed(src_flat, emb_table, pos_table):
    kfn = pl.kernel(
        _body,
        out_type=jax.ShapeDtypeStruct((_N, _D), jnp.float32),
        mesh=plsc.VectorSubcoreMesh(core_axis_name="c", subcore_axis_name="s",
                                    num_cores=_NC, num_subcores=_NS),
        scratch_types=[
            pltpu.VMEM((_NCHUNK, _C), jnp.int32),
            pltpu.VMEM((_C, _D), jnp.float32),
            pltpu.VMEM((_C, _D), jnp.float32),
            pltpu.VMEM((_C, _D), jnp.float32),
            pltpu.VMEM((_C, _D), jnp.float32),
            pltpu.SemaphoreType.DMA,
            pltpu.SemaphoreType.DMA,
            pltpu.SemaphoreType.DMA,
            pltpu.SemaphoreType.DMA,
        ],
    )
    return kfn(src_flat, pos_table, emb_table)


def kernel(src, emb_table, pos_table):
    batch, seq = src.shape
    s4 = src.reshape(_BATCH, _NW, 2, _C)
    src_t = (s4.transpose(1, 2, 0, 3)
             .reshape(_NW, _NCHUNK, _C).astype(jnp.int32))
    out = _embed(src_t, emb_table, pos_table)
    return out.reshape(batch, seq, _D)
